# Initial kernel scaffold; baseline (speedup 1.0000x reference)
import jax
import jax.numpy as jnp
from jax.experimental import pallas as pl


def kernel(x, protos0, protos1, protos2):
    raise NotImplementedError("write your pallas kernel here")


# speedup vs baseline: 1.6016x; 1.6016x over previous
import jax
import jax.numpy as jnp
from jax.experimental import pallas as pl

_NUM_POSITION = 64
_DECLAY = 0.4
_NUM_HIERARCHY = 3
_LAST_P = int(_NUM_POSITION * _DECLAY ** (_NUM_HIERARCHY - 1))


def _reduce_kernel(x_ref, o_ref):
    s = jnp.sum(x_ref[0], axis=1)
    c = s.shape[0] // 2
    m = s.shape[0] // c
    o_ref[0, 0, :] = (s[:c] + s[c:]) * (1.0 / (m * _LAST_P))


def kernel(x, protos0, protos1, protos2):
    N, M, C, T, V = x.shape
    assert M == 2
    xr = x.reshape(N, M * C, T * V)
    out = pl.pallas_call(
        _reduce_kernel,
        grid=(N,),
        in_specs=[pl.BlockSpec((1, M * C, T * V), lambda i: (i, 0, 0))],
        out_specs=pl.BlockSpec((1, 1, C), lambda i: (i, 0, 0)),
        out_shape=jax.ShapeDtypeStruct((N, 1, C), jnp.float32),
    )(xr)
    return out.reshape(N, C)
